# baseline (speedup 1.0000x reference)
import functools

import jax
import jax.numpy as jnp
from jax import lax
from jax.experimental import pallas as pl
from jax.experimental.pallas import tpu as pltpu
from jax.experimental.pallas import tpu_sc as plsc

VOCAB = 1000000
DIM = 100
BATCH = 16384
HIST = 200

_LANES = 16
_ROW_BLK = 40000


_N_STREAMS = 4


def _matvec_body(*refs):
    w_ref = refs[_N_STREAMS]
    for k in range(_N_STREAMS):
        refs[_N_STREAMS + 1 + k][...] = (lax.dot_general(
            w_ref[...], refs[k][...],
            (((1,), (1,)), ((), ())),
            preferred_element_type=jnp.float32,
        ) * (1.0 / HIST))[None]


def _table_matvec(table, weights):
    wt = weights.reshape(1, DIM)
    n_steps = VOCAB // _ROW_BLK
    quarter_blocks = n_steps

    def t_spec(k):
        return pl.BlockSpec(
            (_ROW_BLK // _N_STREAMS, DIM),
            lambda i, k=k: (i + k * quarter_blocks, 0))

    outs = pl.pallas_call(
        _matvec_body,
        grid=(n_steps,),
        in_specs=[t_spec(k) for k in range(_N_STREAMS)]
        + [pl.BlockSpec((1, DIM), lambda i: (0, 0))],
        out_specs=[
            pl.BlockSpec((1, 1, _ROW_BLK // _N_STREAMS), lambda i: (i, 0, 0))
            for _ in range(_N_STREAMS)
        ],
        out_shape=[
            jax.ShapeDtypeStruct(
                (n_steps, 1, _ROW_BLK // _N_STREAMS), jnp.float32)
            for _ in range(_N_STREAMS)
        ],
    )(*([table] * _N_STREAMS + [wt]))
    return jnp.concatenate([o.reshape(-1) for o in outs], axis=0)


def _make_sc_gather_sum():
    nc, ns = 2, 16
    nw = nc * ns
    rows_w = BATCH // nw
    chunk_rows = 128
    n_chunks = rows_w // chunk_rows
    chunk_idx = chunk_rows * HIST

    mesh = plsc.VectorSubcoreMesh(core_axis_name="c", subcore_axis_name="s")
    n_full = HIST // _LANES
    tail_at = HIST - _LANES

    @functools.partial(
        pl.kernel,
        mesh=mesh,
        out_type=jax.ShapeDtypeStruct((BATCH * _LANES,), jnp.float32),
        scratch_types=[
            pltpu.VMEM((chunk_idx,), jnp.int32),
            pltpu.VMEM((chunk_idx,), jnp.float32),
            pltpu.VMEM((rows_w * _LANES,), jnp.float32),
            pltpu.SemaphoreType.DMA,
        ],
    )
    def sc_kernel(idx_hbm, s_hbm, out_hbm, idx_v, vals_v, part_v, sem):
        wid = lax.axis_index("s") * nc + lax.axis_index("c")
        row0 = wid * rows_w
        lane = lax.iota(jnp.int32, _LANES)
        tail_mask = jnp.where(
            lane >= (n_full * _LANES - tail_at),
            jnp.float32(1.0), jnp.float32(0.0))

        def chunk_body(c, carry):
            base = row0 * HIST + c * chunk_idx
            pltpu.sync_copy(idx_hbm.at[pl.ds(base, chunk_idx)], idx_v)
            pltpu.async_copy(s_hbm.at[idx_v], vals_v, sem).wait()

            def row_body(r, carry2):
                rb = r * HIST
                acc = vals_v[pl.ds(rb + tail_at, _LANES)] * tail_mask
                for j in range(n_full):
                    acc = acc + vals_v[pl.ds(rb + j * _LANES, _LANES)]
                part_v[pl.ds((c * chunk_rows + r) * _LANES, _LANES)] = acc
                return carry2

            lax.fori_loop(0, chunk_rows, row_body, 0)
            return carry

        lax.fori_loop(0, n_chunks, chunk_body, 0)
        pltpu.sync_copy(part_v, out_hbm.at[pl.ds(row0 * _LANES, rows_w * _LANES)])

    return sc_kernel


_sc_gather_sum = _make_sc_gather_sum()


def _reduce16_body(p_ref, o_ref):
    o_ref[...] = jnp.sum(p_ref[...], axis=1, keepdims=True)


def _reduce16(part):
    blk = 2048
    return pl.pallas_call(
        _reduce16_body,
        grid=(BATCH // blk,),
        in_specs=[pl.BlockSpec((blk, _LANES), lambda i: (i, 0))],
        out_specs=pl.BlockSpec((blk, 1), lambda i: (i, 0)),
        out_shape=jax.ShapeDtypeStruct((BATCH, 1), jnp.float32),
    )(part)


def kernel(batch_word_idxs, table, weights):
    s = _table_matvec(table, weights).reshape(VOCAB)
    idx_flat = batch_word_idxs.reshape(-1).astype(jnp.int32)
    part = _sc_gather_sum(idx_flat, s).reshape(BATCH, _LANES)
    return _reduce16(part)


# speedup vs baseline: 1.0472x; 1.0472x over previous
import functools

import jax
import jax.numpy as jnp
from jax import lax
from jax.experimental import pallas as pl
from jax.experimental.pallas import tpu as pltpu
from jax.experimental.pallas import tpu_sc as plsc

VOCAB = 1000000
DIM = 100
BATCH = 16384
HIST = 200

_LANES = 16
_ROW_BLK = 40000


_MV_R = 8000
_MV_NCH = VOCAB // _MV_R
_MV_NBUF = 6


def _matvec_body(t_hbm, w_ref, o_ref, bufs, sems):
    def mk(g, b):
        return pltpu.make_async_copy(
            t_hbm.at[pl.ds(g * _MV_R, _MV_R), :], bufs.at[b], sems.at[b])

    for b in range(_MV_NBUF):
        mk(b, b).start()

    def outer(io, carry):
        g0 = io * _MV_NBUF
        for b in range(_MV_NBUF):
            g = g0 + b
            mk(g, b).wait()
            res = lax.dot_general(
                w_ref[...], bufs[b],
                (((1,), (1,)), ((), ())),
                preferred_element_type=jnp.float32,
            ) * (1.0 / HIST)
            o_ref[pl.ds(g, 1), :] = res
            nxt = g + _MV_NBUF

            @pl.when(nxt < _MV_NCH)
            def _():
                mk(nxt, b).start()

        return carry

    lax.fori_loop(0, _MV_NCH // _MV_NBUF, outer, 0)
    rem = _MV_NCH % _MV_NBUF
    for b in range(rem):
        g = (_MV_NCH // _MV_NBUF) * _MV_NBUF + b
        mk(g, b).wait()
        res = lax.dot_general(
            w_ref[...], bufs[b],
            (((1,), (1,)), ((), ())),
            preferred_element_type=jnp.float32,
        ) * (1.0 / HIST)
        o_ref[pl.ds(g, 1), :] = res


def _table_matvec(table, weights):
    wt = weights.reshape(1, DIM)
    out = pl.pallas_call(
        _matvec_body,
        in_specs=[
            pl.BlockSpec(memory_space=pltpu.MemorySpace.HBM),
            pl.BlockSpec((1, DIM), lambda: (0, 0)),
        ],
        out_specs=pl.BlockSpec((_MV_NCH, _MV_R), lambda: (0, 0)),
        out_shape=jax.ShapeDtypeStruct((_MV_NCH, _MV_R), jnp.float32),
        scratch_shapes=[
            pltpu.VMEM((_MV_NBUF, _MV_R, DIM), jnp.float32),
            pltpu.SemaphoreType.DMA((_MV_NBUF,)),
        ],
    )(table, wt)
    return out.reshape(VOCAB)


def _make_sc_gather_sum():
    nc, ns = 2, 16
    nw = nc * ns
    rows_w = BATCH // nw
    chunk_rows = 128
    n_chunks = rows_w // chunk_rows
    chunk_idx = chunk_rows * HIST

    mesh = plsc.VectorSubcoreMesh(core_axis_name="c", subcore_axis_name="s")
    n_full = HIST // _LANES
    tail_at = HIST - _LANES

    @functools.partial(
        pl.kernel,
        mesh=mesh,
        out_type=jax.ShapeDtypeStruct((BATCH * _LANES,), jnp.float32),
        scratch_types=[
            pltpu.VMEM((chunk_idx,), jnp.int32),
            pltpu.VMEM((chunk_idx,), jnp.float32),
            pltpu.VMEM((rows_w * _LANES,), jnp.float32),
            pltpu.SemaphoreType.DMA,
        ],
    )
    def sc_kernel(idx_hbm, s_hbm, out_hbm, idx_v, vals_v, part_v, sem):
        wid = lax.axis_index("s") * nc + lax.axis_index("c")
        row0 = wid * rows_w
        lane = lax.iota(jnp.int32, _LANES)
        tail_mask = jnp.where(
            lane >= (n_full * _LANES - tail_at),
            jnp.float32(1.0), jnp.float32(0.0))

        def chunk_body(c, carry):
            base = row0 * HIST + c * chunk_idx
            pltpu.sync_copy(idx_hbm.at[pl.ds(base, chunk_idx)], idx_v)
            pltpu.async_copy(s_hbm.at[idx_v], vals_v, sem).wait()

            def row_body(r, carry2):
                rb = r * HIST
                acc = vals_v[pl.ds(rb + tail_at, _LANES)] * tail_mask
                for j in range(n_full):
                    acc = acc + vals_v[pl.ds(rb + j * _LANES, _LANES)]
                part_v[pl.ds((c * chunk_rows + r) * _LANES, _LANES)] = acc
                return carry2

            lax.fori_loop(0, chunk_rows, row_body, 0)
            return carry

        lax.fori_loop(0, n_chunks, chunk_body, 0)
        pltpu.sync_copy(part_v, out_hbm.at[pl.ds(row0 * _LANES, rows_w * _LANES)])

    return sc_kernel


_sc_gather_sum = _make_sc_gather_sum()


def _reduce16_body(p_ref, o_ref):
    o_ref[...] = jnp.sum(p_ref[...], axis=1, keepdims=True)


def _reduce16(part):
    blk = 2048
    return pl.pallas_call(
        _reduce16_body,
        grid=(BATCH // blk,),
        in_specs=[pl.BlockSpec((blk, _LANES), lambda i: (i, 0))],
        out_specs=pl.BlockSpec((blk, 1), lambda i: (i, 0)),
        out_shape=jax.ShapeDtypeStruct((BATCH, 1), jnp.float32),
    )(part)


def kernel(batch_word_idxs, table, weights):
    s = _table_matvec(table, weights)
    idx_flat = batch_word_idxs.reshape(-1).astype(jnp.int32)
    part = _sc_gather_sum(idx_flat, s).reshape(BATCH, _LANES)
    return _reduce16(part)


# speedup vs baseline: 1.0579x; 1.0102x over previous
import functools

import jax
import jax.numpy as jnp
from jax import lax
from jax.experimental import pallas as pl
from jax.experimental.pallas import tpu as pltpu
from jax.experimental.pallas import tpu_sc as plsc

VOCAB = 1000000
DIM = 100
BATCH = 16384
HIST = 200

_LANES = 16
_ROW_BLK = 40000


_MV_R = 8000
_MV_NCH = VOCAB // _MV_R
_MV_NBUF = 6


def _matvec_body(t_hbm, w_ref, o_ref, bufs, sems):
    def mk(g, b):
        return pltpu.make_async_copy(
            t_hbm.at[pl.ds(g * _MV_R, _MV_R), :], bufs.at[b], sems.at[b])

    for b in range(_MV_NBUF):
        mk(b, b).start()

    def outer(io, carry):
        g0 = io * _MV_NBUF
        for b in range(_MV_NBUF):
            g = g0 + b
            mk(g, b).wait()
            res = lax.dot_general(
                w_ref[...], bufs[b],
                (((1,), (1,)), ((), ())),
                preferred_element_type=jnp.float32,
            ) * (1.0 / HIST)
            o_ref[pl.ds(g, 1), :] = res
            nxt = g + _MV_NBUF

            @pl.when(nxt < _MV_NCH)
            def _():
                mk(nxt, b).start()

        return carry

    lax.fori_loop(0, _MV_NCH // _MV_NBUF, outer, 0)
    rem = _MV_NCH % _MV_NBUF
    for b in range(rem):
        g = (_MV_NCH // _MV_NBUF) * _MV_NBUF + b
        mk(g, b).wait()
        res = lax.dot_general(
            w_ref[...], bufs[b],
            (((1,), (1,)), ((), ())),
            preferred_element_type=jnp.float32,
        ) * (1.0 / HIST)
        o_ref[pl.ds(g, 1), :] = res


def _table_matvec(table, weights):
    wt = weights.reshape(1, DIM)
    out = pl.pallas_call(
        _matvec_body,
        in_specs=[
            pl.BlockSpec(memory_space=pltpu.MemorySpace.HBM),
            pl.BlockSpec((1, DIM), lambda: (0, 0)),
        ],
        out_specs=pl.BlockSpec((_MV_NCH, _MV_R), lambda: (0, 0)),
        out_shape=jax.ShapeDtypeStruct((_MV_NCH, _MV_R), jnp.float32),
        scratch_shapes=[
            pltpu.VMEM((_MV_NBUF, _MV_R, DIM), jnp.float32),
            pltpu.SemaphoreType.DMA((_MV_NBUF,)),
        ],
    )(table, wt)
    return out.reshape(VOCAB)


def _make_sc_gather_sum():
    nc, ns = 2, 16
    nw = nc * ns
    rows_w = BATCH // nw
    chunk_rows = 128
    n_chunks = rows_w // chunk_rows
    chunk_idx = chunk_rows * HIST

    mesh = plsc.VectorSubcoreMesh(core_axis_name="c", subcore_axis_name="s")
    n_full = HIST // _LANES
    tail_at = HIST - _LANES

    @functools.partial(
        pl.kernel,
        mesh=mesh,
        out_type=jax.ShapeDtypeStruct((BATCH * _LANES,), jnp.float32),
        scratch_types=[
            pltpu.VMEM((chunk_idx,), jnp.int32),
            pltpu.VMEM((chunk_idx,), jnp.int32),
            pltpu.VMEM((chunk_idx,), jnp.float32),
            pltpu.VMEM((chunk_idx,), jnp.float32),
            pltpu.VMEM((rows_w * _LANES,), jnp.float32),
            pltpu.SemaphoreType.DMA((2,)),
            pltpu.SemaphoreType.DMA((2,)),
        ],
    )
    def sc_kernel(idx_hbm, s_hbm, out_hbm, idx_v0, idx_v1, vals_v0, vals_v1,
                  part_v, isem, gsem):
        idx_bufs = (idx_v0, idx_v1)
        vals_bufs = (vals_v0, vals_v1)
        wid = lax.axis_index("s") * nc + lax.axis_index("c")
        row0 = wid * rows_w
        lane = lax.iota(jnp.int32, _LANES)
        tail_mask = jnp.where(
            lane >= (n_full * _LANES - tail_at),
            jnp.float32(1.0), jnp.float32(0.0))

        def idx_copy(c):
            base = row0 * HIST + c * chunk_idx
            return pltpu.async_copy(
                idx_hbm.at[pl.ds(base, chunk_idx)], idx_bufs[c % 2],
                isem.at[c % 2])

        def gather(c):
            return pltpu.async_copy(
                s_hbm.at[idx_bufs[c % 2]], vals_bufs[c % 2], gsem.at[c % 2])

        def fold(c):
            vb = vals_bufs[c % 2]

            def row_body(r, carry2):
                rb = r * HIST
                acc = vb[pl.ds(rb + tail_at, _LANES)] * tail_mask
                for j in range(n_full):
                    acc = acc + vb[pl.ds(rb + j * _LANES, _LANES)]
                part_v[pl.ds((c * chunk_rows + r) * _LANES, _LANES)] = acc
                return carry2

            lax.fori_loop(0, chunk_rows, row_body, 0)

        cps = {0: idx_copy(0), 1: idx_copy(1)}
        cps[0].wait()
        g = gather(0)
        for c in range(n_chunks):
            g.wait()
            if c + 1 < n_chunks:
                cps[c + 1].wait()
                g = gather(c + 1)
            if c + 2 < n_chunks:
                cps[c + 2] = idx_copy(c + 2)
            fold(c)
        pltpu.sync_copy(part_v, out_hbm.at[pl.ds(row0 * _LANES, rows_w * _LANES)])

    return sc_kernel


_sc_gather_sum = _make_sc_gather_sum()


def _reduce16_body(p_ref, o_ref):
    o_ref[...] = jnp.sum(p_ref[...], axis=1, keepdims=True)


def _reduce16(part):
    blk = 2048
    return pl.pallas_call(
        _reduce16_body,
        grid=(BATCH // blk,),
        in_specs=[pl.BlockSpec((blk, _LANES), lambda i: (i, 0))],
        out_specs=pl.BlockSpec((blk, 1), lambda i: (i, 0)),
        out_shape=jax.ShapeDtypeStruct((BATCH, 1), jnp.float32),
    )(part)


def kernel(batch_word_idxs, table, weights):
    s = _table_matvec(table, weights)
    idx_flat = batch_word_idxs.reshape(-1).astype(jnp.int32)
    part = _sc_gather_sum(idx_flat, s).reshape(BATCH, _LANES)
    return _reduce16(part)
